# baseline (speedup 1.0000x reference)
import functools

import jax
import jax.numpy as jnp
from jax import lax
from jax.experimental import pallas as pl
from jax.experimental.pallas import tpu as pltpu
from jax.experimental.pallas import tpu_sc as plsc

DIM = 64
BATCH = 16384

_NC = 2
_NS = 16
_NW = _NC * _NS
_B_PER_W = BATCH // _NW
_CHUNK = 128
_N_CHUNK = _B_PER_W // _CHUNK

_mesh = plsc.VectorSubcoreMesh(core_axis_name="c", subcore_axis_name="s")


@functools.partial(
    pl.kernel,
    mesh=_mesh,
    out_type=jax.ShapeDtypeStruct((BATCH, DIM), jnp.float32),
    scratch_types=[
        pltpu.VMEM((_N_CHUNK, _CHUNK), jnp.int32),
        pltpu.VMEM((_B_PER_W, DIM), jnp.float32),
        pltpu.SemaphoreType.DMA,
    ],
    compiler_params=pltpu.CompilerParams(use_tc_tiling_on_sc=False),
)
def _gather_rows(idx_hbm, table_hbm, out_hbm, idx_v, rows_v, sem):
    wid = lax.axis_index("s") * _NC + lax.axis_index("c")
    pltpu.sync_copy(idx_hbm.at[wid], idx_v)
    copies = []
    for j in range(_N_CHUNK):
        copies.append(
            pltpu.async_copy(
                table_hbm.at[idx_v.at[j]],
                rows_v.at[pl.ds(j * _CHUNK, _CHUNK)],
                sem,
            )
        )
    for c in copies:
        c.wait()
    pltpu.sync_copy(rows_v, out_hbm.at[pl.ds(wid * _B_PER_W, _B_PER_W)])


def kernel(ind, z):
    if ind.ndim == 0:
        ind = ind.reshape((1,))
    idx3 = ind.reshape(_NW, _N_CHUNK, _CHUNK)
    out = _gather_rows(idx3, z)
    return out.reshape(ind.shape[0], 1, DIM)


# speedup vs baseline: 1.6280x; 1.6280x over previous
import functools

import jax
import jax.numpy as jnp
from jax import lax
from jax.experimental import pallas as pl
from jax.experimental.pallas import tpu as pltpu
from jax.experimental.pallas import tpu_sc as plsc

DIM = 64
BATCH = 16384

_NC = 2
_NS = 16
_NW = _NC * _NS
_B_PER_W = BATCH // _NW
_CHUNK = 32
_N_CHUNK = _B_PER_W // _CHUNK

_mesh = plsc.VectorSubcoreMesh(core_axis_name="c", subcore_axis_name="s")


@functools.partial(
    pl.kernel,
    mesh=_mesh,
    out_type=jax.ShapeDtypeStruct((BATCH, 1, DIM), jnp.float32),
    scratch_types=[
        pltpu.VMEM((_B_PER_W,), jnp.int32),
        pltpu.VMEM((_CHUNK, DIM), jnp.float32),
        pltpu.SemaphoreType.DMA,
    ],
)
def _gather_rows(idx_hbm, tab_hbm, out_hbm, idx_v, sel_v, sem):
    wid = lax.axis_index("s") * _NC + lax.axis_index("c")
    base = wid * _B_PER_W
    pltpu.sync_copy(idx_hbm.at[pl.ds(base, _B_PER_W)], idx_v)

    def chunk_body(c, _):
        copies = []
        for g in range(_CHUNK // 16):
            vec = idx_v[pl.ds(c * _CHUNK + g * 16, 16)]
            for i in range(16):
                r = lax.squeeze(lax.slice(vec, (i,), (i + 1,)), (0,))
                copies.append(
                    pltpu.async_copy(
                        tab_hbm.at[r], sel_v.at[g * 16 + i], sem
                    )
                )
        for cp in copies:
            cp.wait()
        pltpu.sync_copy(
            sel_v, out_hbm.at[pl.ds(base + c * _CHUNK, _CHUNK), 0, :]
        )
        return ()

    lax.fori_loop(0, _N_CHUNK, chunk_body, (), unroll=False)


def kernel(ind, z):
    if ind.ndim == 0:
        ind = ind.reshape((1,))
    return _gather_rows(ind, z)


# speedup vs baseline: 1.6317x; 1.0023x over previous
import functools

import jax
import jax.numpy as jnp
from jax import lax
from jax.experimental import pallas as pl
from jax.experimental.pallas import tpu as pltpu
from jax.experimental.pallas import tpu_sc as plsc

DIM = 64
BATCH = 16384

_NC = 2
_NS = 16
_NW = _NC * _NS
_B_PER_W = BATCH // _NW
_CHUNK = 32
_N_CHUNK = _B_PER_W // _CHUNK

_mesh = plsc.VectorSubcoreMesh(core_axis_name="c", subcore_axis_name="s")


@functools.partial(
    pl.kernel,
    mesh=_mesh,
    out_type=jax.ShapeDtypeStruct((BATCH, 1, DIM), jnp.float32),
    scratch_types=[
        pltpu.VMEM((_B_PER_W,), jnp.int32),
        pltpu.VMEM((_CHUNK, DIM), jnp.float32),
        pltpu.SemaphoreType.DMA,
    ],
    compiler_params=pltpu.CompilerParams(use_tc_tiling_on_sc=True),
)
def _gather_rows(idx_hbm, tab_hbm, out_hbm, idx_v, sel_v, sem):
    wid = lax.axis_index("s") * _NC + lax.axis_index("c")
    base = wid * _B_PER_W
    pltpu.sync_copy(idx_hbm.at[pl.ds(base, _B_PER_W)], idx_v)

    def chunk_body(c, _):
        copies = []
        for g in range(_CHUNK // 16):
            vec = idx_v[pl.ds(c * _CHUNK + g * 16, 16)]
            for i in range(16):
                r = lax.squeeze(lax.slice(vec, (i,), (i + 1,)), (0,))
                copies.append(
                    pltpu.async_copy(
                        tab_hbm.at[r], sel_v.at[g * 16 + i], sem
                    )
                )
        for cp in copies:
            cp.wait()
        pltpu.sync_copy(
            sel_v, out_hbm.at[pl.ds(base + c * _CHUNK, _CHUNK), 0, :]
        )
        return ()

    lax.fori_loop(0, _N_CHUNK, chunk_body, (), unroll=False)


def kernel(ind, z):
    if ind.ndim == 0:
        ind = ind.reshape((1,))
    return _gather_rows(ind, z)
